# baseline (speedup 1.0000x reference)
import functools

import jax
import jax.numpy as jnp
from jax import lax
from jax.experimental import pallas as pl
from jax.experimental.pallas import tpu as pltpu
from jax.experimental.pallas import tpu_sc as plsc

NUM_ENT = 1000000
NUM_REL = 1000
EMB_DIM = 64
BATCH = 16384

_INFO = plsc.get_sparse_core_info()
_NC = _INFO.num_cores
_NS = _INFO.num_subcores
_NW = _NC * _NS
_BPW = BATCH // _NW
_LANES = 16
_CHUNKS = _BPW // _LANES
_QLEN = _BPW * _LANES


def _dist_mult_body(sub_h, rel_h, obj_h, ent_h, rele_h, bias_h, out_h,
                    sidx, ridx, oidx, h_v, r_v, t_v, b_v, out_v,
                    q_v, scat_idx, acc_v, zeros_v, shared_acc, sem):
    wid = lax.axis_index("s") * _NC + lax.axis_index("c")
    sid = lax.axis_index("s")
    base = wid * _BPW

    pltpu.sync_copy(sub_h.at[pl.ds(base, _BPW)], sidx)
    pltpu.sync_copy(rel_h.at[pl.ds(base, _BPW)], ridx)
    pltpu.sync_copy(obj_h.at[pl.ds(base, _BPW)], oidx)

    c1 = pltpu.async_copy(ent_h.at[sidx], h_v, sem)
    c2 = pltpu.async_copy(rele_h.at[ridx], r_v, sem)
    c3 = pltpu.async_copy(ent_h.at[oidx], t_v, sem)
    c4 = pltpu.async_copy(bias_h.at[oidx], b_v, sem)

    zero = jnp.zeros((_LANES,), jnp.float32)

    def fill(i, carry):
        scat_idx[pl.ds(i * _LANES, _LANES)] = jnp.full(
            (_LANES,), sid * _BPW, jnp.int32) + i
        return carry

    lax.fori_loop(0, _BPW, fill, 0)
    for i in range(_BPW // _LANES):
        zeros_v[pl.ds(i * _LANES, _LANES)] = zero
    pltpu.sync_copy(zeros_v, shared_acc.at[pl.ds(sid * _BPW, _BPW)])

    c1.wait()
    c2.wait()
    c3.wait()
    c4.wait()

    def chunk(j, carry):
        for l in range(_LANES):
            row = j * _LANES + l
            sl0 = pl.ds(0, _LANES)
            sl1 = pl.ds(_LANES, _LANES)
            sl2 = pl.ds(2 * _LANES, _LANES)
            sl3 = pl.ds(3 * _LANES, _LANES)
            q = (h_v[row, sl0] * r_v[row, sl0] * t_v[row, sl0]
                 + h_v[row, sl1] * r_v[row, sl1] * t_v[row, sl1]
                 + h_v[row, sl2] * r_v[row, sl2] * t_v[row, sl2]
                 + h_v[row, sl3] * r_v[row, sl3] * t_v[row, sl3])
            q_v[pl.ds(row * _LANES, _LANES)] = q
        return carry

    lax.fori_loop(0, _CHUNKS, chunk, 0)

    pltpu.sync_copy(q_v, shared_acc.at[scat_idx], add=True)
    pltpu.sync_copy(shared_acc.at[pl.ds(sid * _BPW, _BPW)], acc_v)

    for j in range(_CHUNKS):
        sl = pl.ds(j * _LANES, _LANES)
        score = acc_v[sl] + b_v[sl]
        out_v[sl] = 1.0 / (1.0 + jnp.exp(-score))

    pltpu.sync_copy(out_v, out_h.at[pl.ds(base, _BPW)])


@jax.jit
def kernel(sub, rel, obj, ent_emb, rel_emb, bias):
    mesh = plsc.VectorSubcoreMesh(core_axis_name="c", subcore_axis_name="s")
    k = functools.partial(
        pl.kernel,
        mesh=mesh,
        out_type=jax.ShapeDtypeStruct((BATCH,), jnp.float32),
        compiler_params=pltpu.CompilerParams(use_tc_tiling_on_sc=False),
        scratch_types=[
            pltpu.VMEM((_BPW,), jnp.int32),
            pltpu.VMEM((_BPW,), jnp.int32),
            pltpu.VMEM((_BPW,), jnp.int32),
            pltpu.VMEM((_BPW, EMB_DIM), jnp.float32),
            pltpu.VMEM((_BPW, EMB_DIM), jnp.float32),
            pltpu.VMEM((_BPW, EMB_DIM), jnp.float32),
            pltpu.VMEM((_BPW,), jnp.float32),
            pltpu.VMEM((_BPW,), jnp.float32),
            pltpu.VMEM((_QLEN,), jnp.float32),
            pltpu.VMEM((_QLEN,), jnp.int32),
            pltpu.VMEM((_BPW,), jnp.float32),
            pltpu.VMEM((_BPW,), jnp.float32),
            pltpu.VMEM_SHARED((_NS * _BPW,), jnp.float32),
            pltpu.SemaphoreType.DMA,
        ],
    )(_dist_mult_body)
    return k(sub.astype(jnp.int32), rel.astype(jnp.int32),
             obj.astype(jnp.int32), ent_emb, rel_emb, bias)
